# Initial kernel scaffold; baseline (speedup 1.0000x reference)
import functools

import jax
import jax.numpy as jnp
from jax import lax
from jax.experimental import pallas as pl
from jax.experimental.pallas import tpu as pltpu
from jax.experimental.pallas import tpu_sc as plsc

NE = 8192
ED = 256
B = 16
T = 1024
NROW = B * T
BETA = 0.25
ALPHA = 1.0

ROW_TILE = 512
CB_CHUNK = 1024

NW = 32
BPW = NROW // NW
GCH = 128
NCH = BPW // GCH


def _argmin_body(ze_ref, cb_ref, idx_ref, loss_ref):
    i = pl.program_id(0)
    zb = ze_ref[0]
    nk = NE // CB_CHUNK

    def chunk(k, carry):
        rmin, ridx = carry
        cb = cb_ref[pl.ds(k * CB_CHUNK, CB_CHUNK), :]
        c2 = jnp.sum(cb * cb, axis=1)
        m = lax.dot_general(zb, cb, (((0,), (1,)), ((), ())),
                            preferred_element_type=jnp.float32)
        scores = c2[None, :] - 2.0 * m
        mn = jnp.min(scores, axis=1)
        io = lax.broadcasted_iota(jnp.int32, scores.shape, 1) + k * CB_CHUNK
        am = jnp.min(jnp.where(scores == mn[:, None], io, jnp.int32(2**30)),
                     axis=1)
        better = mn < rmin
        return jnp.where(better, mn, rmin), jnp.where(better, am, ridx)

    init = (jnp.full((ROW_TILE,), jnp.inf, jnp.float32),
            jnp.zeros((ROW_TILE,), jnp.int32))
    rmin, ridx = lax.fori_loop(0, nk, chunk, init)
    idx_ref[0, 0, :] = ridx
    z2 = jnp.sum(zb * zb, axis=0)
    part = jnp.sum(rmin) + jnp.sum(z2)

    @pl.when(i == 0)
    def _():
        loss_ref[0, 0] = 0.0

    loss_ref[0, 0] += part


_argmin_call = pl.pallas_call(
    _argmin_body,
    grid=(NROW // ROW_TILE,),
    in_specs=[
        pl.BlockSpec((1, ED, ROW_TILE),
                     lambda i: (i // (T // ROW_TILE), 0, i % (T // ROW_TILE))),
        pl.BlockSpec((NE, ED), lambda i: (0, 0)),
    ],
    out_specs=[
        pl.BlockSpec((1, 1, ROW_TILE), lambda i: (i, 0, 0)),
        pl.BlockSpec((1, 1), lambda i: (0, 0), memory_space=pltpu.SMEM),
    ],
    out_shape=[
        jax.ShapeDtypeStruct((NROW // ROW_TILE, 1, ROW_TILE), jnp.int32),
        jax.ShapeDtypeStruct((1, 1), jnp.float32),
    ],
)


def _sc_body(cb_hbm, idx_hbm, zq_hbm, counts_hbm,
             idx_v, rows_v, ones_v, slab_v, shared_counts, sem):
    cid = lax.axis_index("c")
    sid = lax.axis_index("s")
    wid = sid * 2 + cid
    pltpu.sync_copy(idx_hbm.at[pl.ds(wid * NCH, NCH)], idx_v)

    for j in range(BPW // 16):
        slab_v[pl.ds(j * 16, 16)] = jnp.zeros((16,), jnp.float32)
    pltpu.sync_copy(slab_v, shared_counts.at[pl.ds(sid * BPW, BPW)])
    for j in range(GCH // 16):
        ones_v[pl.ds(j * 16, 16)] = jnp.ones((16,), jnp.float32)
    plsc.subcore_barrier()

    for j in range(NCH):
        pltpu.sync_copy(ones_v, shared_counts.at[idx_v.at[j]], add=True)

    for j in range(NCH):
        pltpu.async_copy(cb_hbm.at[idx_v.at[j]], rows_v, sem).wait()
        pltpu.sync_copy(rows_v, zq_hbm.at[pl.ds(wid * BPW + j * GCH, GCH)])

    plsc.subcore_barrier()
    pltpu.sync_copy(shared_counts.at[pl.ds(sid * BPW, BPW)], slab_v)
    pltpu.sync_copy(slab_v, counts_hbm.at[cid, pl.ds(sid * BPW, BPW)])


_sc_call = pl.kernel(
    _sc_body,
    mesh=plsc.VectorSubcoreMesh(core_axis_name="c", subcore_axis_name="s"),
    out_type=[
        jax.ShapeDtypeStruct((NROW, ED), jnp.float32),
        jax.ShapeDtypeStruct((2, NE), jnp.float32),
    ],
    scratch_types=[
        pltpu.VMEM((NCH, GCH), jnp.int32),
        pltpu.VMEM((GCH, ED), jnp.float32),
        pltpu.VMEM((GCH,), jnp.float32),
        pltpu.VMEM((BPW,), jnp.float32),
        pltpu.VMEM_SHARED((NE,), jnp.float32),
        pltpu.SemaphoreType.DMA,
    ],
)


def _finish_body(zq_ref, counts_ref, loss_ref, out_ref, vq_ref, perp_ref):
    i = pl.program_id(0)
    out_ref[0] = jnp.transpose(zq_ref[0], (1, 0))

    @pl.when(i == 0)
    def _():
        tot = counts_ref[0, :] + counts_ref[1, :]
        p = tot * (1.0 / NROW)
        ent = jnp.sum(p * jnp.log(p + 1e-10))
        perp_ref[0, 0] = jnp.exp(-ent)
        vq_ref[0, 0] = loss_ref[0, 0] * (ALPHA / (NROW * ED))


_finish_call = pl.pallas_call(
    _finish_body,
    grid=(B,),
    in_specs=[
        pl.BlockSpec((1, T, ED), lambda i: (i, 0, 0)),
        pl.BlockSpec((2, NE), lambda i: (0, 0)),
        pl.BlockSpec((1, 1), lambda i: (0, 0), memory_space=pltpu.SMEM),
    ],
    out_specs=[
        pl.BlockSpec((1, ED, T), lambda i: (i, 0, 0)),
        pl.BlockSpec((1, 1), lambda i: (0, 0), memory_space=pltpu.SMEM),
        pl.BlockSpec((1, 1), lambda i: (0, 0), memory_space=pltpu.SMEM),
    ],
    out_shape=[
        jax.ShapeDtypeStruct((B, ED, T), jnp.float32),
        jax.ShapeDtypeStruct((1, 1), jnp.float32),
        jax.ShapeDtypeStruct((1, 1), jnp.float32),
    ],
)


def kernel(z_e, codebook):
    idx3, loss = _argmin_call(z_e, codebook)
    idx2 = idx3.reshape(NROW // GCH, GCH)
    zq_flat, counts = _sc_call(codebook, idx2)
    zq3 = zq_flat.reshape(B, T, ED)
    zq_out, vq, perp = _finish_call(zq3, counts, loss)
    return zq_out, vq.reshape(()), perp.reshape(())


# speedup vs baseline: 1.2960x; 1.2960x over previous
import functools

import jax
import jax.numpy as jnp
from jax import lax
from jax.experimental import pallas as pl
from jax.experimental.pallas import tpu as pltpu
from jax.experimental.pallas import tpu_sc as plsc

NE = 8192
ED = 256
B = 16
T = 1024
NROW = B * T
BETA = 0.25
ALPHA = 1.0

ROW_TILE = 512
CB_CHUNK = 1024

NW = 32
BPW = NROW // NW
GCH = 128
NCH = BPW // GCH


NK = NE // CB_CHUNK


def _argmin_body(ze_ref, cb_ref, idx_ref, loss_ref, rmin_s, ridx_s,
                 dot_kwargs=None):
    i = pl.program_id(0)
    k = pl.program_id(1)
    zb = ze_ref[0]
    cb = cb_ref[...]
    c2 = jnp.sum(cb * cb, axis=1, keepdims=True)
    m = lax.dot_general(cb, zb, (((1,), (0,)), ((), ())),
                        preferred_element_type=jnp.float32,
                        **(dot_kwargs or {}))
    scores = c2 - 2.0 * m
    mn = jnp.min(scores, axis=0, keepdims=True)
    io = lax.broadcasted_iota(jnp.int32, scores.shape, 0) + k * CB_CHUNK
    am = jnp.min(jnp.where(scores == mn, io, jnp.int32(2**30)),
                 axis=0, keepdims=True)

    @pl.when(k == 0)
    def _():
        rmin_s[...] = mn
        ridx_s[...] = am

    @pl.when(k > 0)
    def _():
        better = mn < rmin_s[...]
        rmin_s[...] = jnp.where(better, mn, rmin_s[...])
        ridx_s[...] = jnp.where(better, am, ridx_s[...])

    @pl.when(k == NK - 1)
    def _():
        idx_ref[0, 0, :] = ridx_s[0, :]
        part = jnp.sum(rmin_s[...]) + jnp.sum(zb * zb)

        @pl.when(i == 0)
        def _():
            loss_ref[0, 0] = 0.0

        loss_ref[0, 0] += part


def _make_argmin_call(dot_kwargs=None):
    return pl.pallas_call(
        functools.partial(_argmin_body, dot_kwargs=dot_kwargs),
        grid=(NROW // ROW_TILE, NK),
        in_specs=[
            pl.BlockSpec((1, ED, ROW_TILE),
                         lambda i, k: (i // (T // ROW_TILE), 0,
                                       i % (T // ROW_TILE))),
            pl.BlockSpec((CB_CHUNK, ED), lambda i, k: (k, 0)),
        ],
        out_specs=[
            pl.BlockSpec((1, 1, ROW_TILE), lambda i, k: (i, 0, 0)),
            pl.BlockSpec((1, 1), lambda i, k: (0, 0), memory_space=pltpu.SMEM),
        ],
        out_shape=[
            jax.ShapeDtypeStruct((NROW // ROW_TILE, 1, ROW_TILE), jnp.int32),
            jax.ShapeDtypeStruct((1, 1), jnp.float32),
        ],
        scratch_shapes=[
            pltpu.VMEM((1, ROW_TILE), jnp.float32),
            pltpu.VMEM((1, ROW_TILE), jnp.int32),
        ],
    )


_argmin_call = _make_argmin_call(None)


def _sc_body(cb_hbm, idx_hbm, zq_hbm, counts_hbm,
             idx_v, rows_v, ones_v, slab_v, shared_counts, sem):
    cid = lax.axis_index("c")
    sid = lax.axis_index("s")
    wid = sid * 2 + cid
    pltpu.sync_copy(idx_hbm.at[pl.ds(wid * NCH, NCH)], idx_v)

    for j in range(BPW // 16):
        slab_v[pl.ds(j * 16, 16)] = jnp.zeros((16,), jnp.float32)
    pltpu.sync_copy(slab_v, shared_counts.at[pl.ds(sid * BPW, BPW)])
    for j in range(GCH // 16):
        ones_v[pl.ds(j * 16, 16)] = jnp.ones((16,), jnp.float32)
    plsc.subcore_barrier()

    for j in range(NCH):
        pltpu.sync_copy(ones_v, shared_counts.at[idx_v.at[j]], add=True)

    for j in range(NCH):
        pltpu.async_copy(cb_hbm.at[idx_v.at[j]], rows_v, sem).wait()
        pltpu.sync_copy(rows_v, zq_hbm.at[pl.ds(wid * BPW + j * GCH, GCH)])

    plsc.subcore_barrier()
    pltpu.sync_copy(shared_counts.at[pl.ds(sid * BPW, BPW)], slab_v)
    pltpu.sync_copy(slab_v, counts_hbm.at[cid, pl.ds(sid * BPW, BPW)])


@functools.lru_cache(maxsize=1)
def _get_sc_call():
    return pl.kernel(
        _sc_body,
        mesh=plsc.VectorSubcoreMesh(core_axis_name="c", subcore_axis_name="s"),
        out_type=[
            jax.ShapeDtypeStruct((NROW, ED), jnp.float32),
            jax.ShapeDtypeStruct((2, NE), jnp.float32),
        ],
        scratch_types=[
            pltpu.VMEM((NCH, GCH), jnp.int32),
            pltpu.VMEM((GCH, ED), jnp.float32),
            pltpu.VMEM((GCH,), jnp.float32),
            pltpu.VMEM((BPW,), jnp.float32),
            pltpu.VMEM_SHARED((NE,), jnp.float32),
            pltpu.SemaphoreType.DMA,
        ],
    )


def _finish_body(zq_ref, counts_ref, loss_ref, out_ref, vq_ref, perp_ref):
    i = pl.program_id(0)
    out_ref[0] = jnp.transpose(zq_ref[0], (1, 0))

    @pl.when(i == 0)
    def _():
        tot = counts_ref[0, :] + counts_ref[1, :]
        p = tot * (1.0 / NROW)
        ent = jnp.sum(p * jnp.log(p + 1e-10))
        perp_ref[0, 0] = jnp.exp(-ent)
        vq_ref[0, 0] = loss_ref[0, 0] * (ALPHA / (NROW * ED))


_finish_call = pl.pallas_call(
    _finish_body,
    grid=(B,),
    in_specs=[
        pl.BlockSpec((1, T, ED), lambda i: (i, 0, 0)),
        pl.BlockSpec((2, NE), lambda i: (0, 0)),
        pl.BlockSpec((1, 1), lambda i: (0, 0), memory_space=pltpu.SMEM),
    ],
    out_specs=[
        pl.BlockSpec((1, ED, T), lambda i: (i, 0, 0)),
        pl.BlockSpec((1, 1), lambda i: (0, 0), memory_space=pltpu.SMEM),
        pl.BlockSpec((1, 1), lambda i: (0, 0), memory_space=pltpu.SMEM),
    ],
    out_shape=[
        jax.ShapeDtypeStruct((B, ED, T), jnp.float32),
        jax.ShapeDtypeStruct((1, 1), jnp.float32),
        jax.ShapeDtypeStruct((1, 1), jnp.float32),
    ],
)


def kernel(z_e, codebook):
    idx3, loss = _argmin_call(z_e, codebook)
    idx2 = idx3.reshape(NROW // GCH, GCH)
    zq_flat, counts = _get_sc_call()(codebook, idx2)
    zq3 = zq_flat.reshape(B, T, ED)
    zq_out, vq, perp = _finish_call(zq3, counts, loss)
    return zq_out, vq.reshape(()), perp.reshape(())
